# Initial kernel scaffold; baseline (speedup 1.0000x reference)
import functools
import math

import jax
import jax.numpy as jnp
from jax import lax
from jax.experimental import pallas as pl
from jax.experimental.pallas import tpu as pltpu
from jax.experimental.pallas import tpu_sc as plsc

_NCG = 4


def _emb_body(nbg, cpg, scale, tPT_hbm, tFT_hbm, phon_hbm, a1_hbm, f2_hbm,
              out_hbm, tP_ref, tF_ref, idxp_ref, idxf_ref, a1_ref, out_ref):
    B, L = phon_hbm.shape
    C = tPT_hbm.shape[0]
    nc = plsc.get_sparse_core_info().num_cores
    wid = lax.axis_index("s") * nc + lax.axis_index("c")
    cg = wid % _NCG
    bg = wid // _NCG
    nb = B // nbg
    b0 = bg * nb
    c0 = cg * cpg

    pltpu.sync_copy(tPT_hbm.at[pl.ds(c0, cpg), :], tP_ref)
    pltpu.sync_copy(tFT_hbm.at[pl.ds(c0, cpg), :], tF_ref)

    nk = (L + 15) // 16

    def bbody(i, carry):
        b = b0 + i
        pltpu.sync_copy(phon_hbm.at[b], idxp_ref)
        pltpu.sync_copy(f2_hbm.at[b], idxf_ref)
        pltpu.sync_copy(a1_hbm.at[b], a1_ref)

        def kbody(k, kcarry):
            s = pl.multiple_of(jnp.minimum(k * 16, L - 16), 8)
            ip = idxp_ref[pl.ds(s, 16)]
            iff = idxf_ref[pl.ds(s, 16)]
            av = a1_ref[pl.ds(s, 16)]
            for c in range(cpg):
                cv = jnp.full((16,), c, jnp.int32)
                out_ref[c, pl.ds(s, 16)] = plsc.load_gather(tP_ref, [cv, ip]) * scale
                out_ref[cpg + c, pl.ds(s, 16)] = plsc.load_gather(tF_ref, [cv, iff]) * scale
                out_ref[2 * cpg + c, pl.ds(s, 16)] = av
            return kcarry

        lax.fori_loop(0, nk, kbody, 0)

        pltpu.sync_copy(out_ref.at[pl.ds(0, cpg), :], out_hbm.at[b, pl.ds(c0, cpg), :])
        pltpu.sync_copy(out_ref.at[pl.ds(cpg, cpg), :], out_hbm.at[b, pl.ds(C + c0, cpg), :])
        pltpu.sync_copy(out_ref.at[pl.ds(2 * cpg, cpg), :], out_hbm.at[b, pl.ds(2 * C + c0, cpg), :])
        return carry

    lax.fori_loop(0, nb, bbody, 0)


def kernel(phoneme, a1, f2, phoneme_table, f2_table):
    B, L = phoneme.shape
    V, C = phoneme_table.shape
    scale = math.sqrt(C)
    info = plsc.get_sparse_core_info()
    nw = info.num_cores * info.num_subcores
    nbg = nw // _NCG
    cpg = C // _NCG
    assert C % _NCG == 0 and B % nbg == 0 and L % 8 == 0 and L >= 16

    tPT = jnp.transpose(phoneme_table)
    tFT = jnp.transpose(f2_table)
    phoneme = phoneme.astype(jnp.int32)
    f2 = f2.astype(jnp.int32)
    a1 = a1.astype(jnp.float32)

    mesh = plsc.VectorSubcoreMesh(core_axis_name="c", subcore_axis_name="s")
    run = pl.kernel(
        functools.partial(_emb_body, nbg, cpg, scale),
        out_type=jax.ShapeDtypeStruct((B, 3 * C, L), jnp.float32),
        mesh=mesh,
        scratch_types=[
            pltpu.VMEM((cpg, V), jnp.float32),
            pltpu.VMEM((cpg, V), jnp.float32),
            pltpu.VMEM((L,), jnp.int32),
            pltpu.VMEM((L,), jnp.int32),
            pltpu.VMEM((L,), jnp.float32),
            pltpu.VMEM((3 * cpg, L), jnp.float32),
        ],
    )
    return run(tPT, tFT, phoneme, a1, f2)


# speedup vs baseline: 1.0208x; 1.0208x over previous
import functools
import math

import jax
import jax.numpy as jnp
from jax import lax
from jax.experimental import pallas as pl
from jax.experimental.pallas import tpu as pltpu
from jax.experimental.pallas import tpu_sc as plsc

_NCG = 4


def _emb_body(dims, scale, tPT_hbm, tFT_hbm, phon_hbm, a1_hbm, f2_hbm,
              out_hbm, tP_ref, tF_ref, idxp_ref, idxf_ref, a1_ref, out_ref):
    B, L, V, C, nbg, cpg = dims
    nc = plsc.get_sparse_core_info().num_cores
    wid = lax.axis_index("s") * nc + lax.axis_index("c")
    cg = wid % _NCG
    bg = wid // _NCG
    nb = B // nbg
    b0 = bg * nb
    c0 = cg * cpg

    pltpu.sync_copy(tPT_hbm.at[pl.ds(c0 * V, cpg * V)], tP_ref)
    pltpu.sync_copy(tFT_hbm.at[pl.ds(c0 * V, cpg * V)], tF_ref)

    nk = (L + 15) // 16

    def bbody(i, carry):
        b = b0 + i
        pltpu.sync_copy(phon_hbm.at[pl.ds(b * L, L)], idxp_ref)
        pltpu.sync_copy(f2_hbm.at[pl.ds(b * L, L)], idxf_ref)
        pltpu.sync_copy(a1_hbm.at[pl.ds(b * L, L)], a1_ref)

        def kbody(k, kcarry):
            s = pl.multiple_of(jnp.minimum(k * 16, L - 16), 8)
            ip = idxp_ref[pl.ds(s, 16)]
            iff = idxf_ref[pl.ds(s, 16)]
            av = a1_ref[pl.ds(s, 16)]
            for c in range(cpg):
                out_ref[pl.ds(c * L + s, 16)] = (
                    plsc.load_gather(tP_ref, [ip + c * V]) * scale)
                out_ref[pl.ds((cpg + c) * L + s, 16)] = (
                    plsc.load_gather(tF_ref, [iff + c * V]) * scale)
                out_ref[pl.ds((2 * cpg + c) * L + s, 16)] = av
            return kcarry

        lax.fori_loop(0, nk, kbody, 0)

        obase = b * 3 * C * L
        pltpu.sync_copy(out_ref.at[pl.ds(0, cpg * L)],
                        out_hbm.at[pl.ds(obase + c0 * L, cpg * L)])
        pltpu.sync_copy(out_ref.at[pl.ds(cpg * L, cpg * L)],
                        out_hbm.at[pl.ds(obase + (C + c0) * L, cpg * L)])
        pltpu.sync_copy(out_ref.at[pl.ds(2 * cpg * L, cpg * L)],
                        out_hbm.at[pl.ds(obase + (2 * C + c0) * L, cpg * L)])
        return carry

    lax.fori_loop(0, nb, bbody, 0)


def kernel(phoneme, a1, f2, phoneme_table, f2_table):
    B, L = phoneme.shape
    V, C = phoneme_table.shape
    scale = math.sqrt(C)
    info = plsc.get_sparse_core_info()
    nw = info.num_cores * info.num_subcores
    nbg = nw // _NCG
    cpg = C // _NCG
    assert C % _NCG == 0 and B % nbg == 0 and L % 8 == 0 and L >= 16

    tPT = jnp.transpose(phoneme_table).reshape(-1)
    tFT = jnp.transpose(f2_table).reshape(-1)
    phoneme = phoneme.astype(jnp.int32).reshape(-1)
    f2 = f2.astype(jnp.int32).reshape(-1)
    a1 = a1.astype(jnp.float32).reshape(-1)

    mesh = plsc.VectorSubcoreMesh(core_axis_name="c", subcore_axis_name="s")
    run = pl.kernel(
        functools.partial(_emb_body, (B, L, V, C, nbg, cpg), scale),
        out_type=jax.ShapeDtypeStruct((B * 3 * C * L,), jnp.float32),
        mesh=mesh,
        compiler_params=pltpu.CompilerParams(needs_layout_passes=False),
        scratch_types=[
            pltpu.VMEM((cpg * V,), jnp.float32),
            pltpu.VMEM((cpg * V,), jnp.float32),
            pltpu.VMEM((L,), jnp.int32),
            pltpu.VMEM((L,), jnp.int32),
            pltpu.VMEM((L,), jnp.float32),
            pltpu.VMEM((3 * cpg * L,), jnp.float32),
        ],
    )
    out = run(tPT, tFT, phoneme, a1, f2)
    return out.reshape(B, 3 * C, L)


# speedup vs baseline: 1.2434x; 1.2180x over previous
import functools
import math

import jax
import jax.numpy as jnp
from jax import lax
from jax.experimental import pallas as pl
from jax.experimental.pallas import tpu as pltpu
from jax.experimental.pallas import tpu_sc as plsc

_NCG = 4
_G = 8


def _emb_body(dims, scale, tPT_hbm, tFT_hbm, phon_hbm, a1_hbm, f2_hbm,
              out_hbm, tP_ref, tF_ref, ip0, ip1, if0, if1, ia0, ia1,
              ob0, ob1, is0, is1, os0, os1):
    B, L, V, C, nbg, cpg = dims
    ipb, ifb, iab = [ip0, ip1], [if0, if1], [ia0, ia1]
    obb, isem, osem = [ob0, ob1], [is0, is1], [os0, os1]
    nc = plsc.get_sparse_core_info().num_cores
    wid = lax.axis_index("s") * nc + lax.axis_index("c")
    cg = wid % _NCG
    bg = wid // _NCG
    nb = B // nbg
    ng = nb // _G
    b0 = bg * nb
    c0 = cg * cpg
    nk = (L + 15) // 16
    sec = cpg * L

    pltpu.sync_copy(tPT_hbm.at[pl.ds(c0 * V, cpg * V)], tP_ref)
    pltpu.sync_copy(tFT_hbm.at[pl.ds(c0 * V, cpg * V)], tF_ref)

    def fire_in(g, par):
        gb = (b0 + g * _G) * L
        pltpu.async_copy(phon_hbm.at[pl.ds(gb, _G * L)], ipb[par], isem[par])
        pltpu.async_copy(f2_hbm.at[pl.ds(gb, _G * L)], ifb[par], isem[par])
        pltpu.async_copy(a1_hbm.at[pl.ds(gb, _G * L)], iab[par], isem[par])

    def drain_in(par):
        pltpu.make_async_copy(phon_hbm.at[pl.ds(0, _G * L)], ipb[par], isem[par]).wait()
        pltpu.make_async_copy(f2_hbm.at[pl.ds(0, _G * L)], ifb[par], isem[par]).wait()
        pltpu.make_async_copy(a1_hbm.at[pl.ds(0, _G * L)], iab[par], isem[par]).wait()

    def drain_out(p):
        pltpu.make_async_copy(out_hbm.at[pl.ds(0, 3 * sec)], obb[p], osem[p]).wait()

    def fire_out(b, p):
        obase = b * (3 * C * L)
        ob = obb[p]
        pltpu.async_copy(ob.at[pl.ds(0, sec)],
                         out_hbm.at[pl.ds(obase + c0 * L, sec)], osem[p])
        pltpu.async_copy(ob.at[pl.ds(sec, sec)],
                         out_hbm.at[pl.ds(obase + (C + c0) * L, sec)], osem[p])
        pltpu.async_copy(ob.at[pl.ds(2 * sec, sec)],
                         out_hbm.at[pl.ds(obase + (2 * C + c0) * L, sec)], osem[p])

    def compute(jofs, in_par, p):
        ipr, ifr, iar, ob = ipb[in_par], ifb[in_par], iab[in_par], obb[p]

        def kbody(k, kc):
            s = pl.multiple_of(jnp.minimum(k * 16, L - 16), 8)
            src = pl.multiple_of(jofs + s, 8)
            ipv = ipr[pl.ds(src, 16)]
            ifv = ifr[pl.ds(src, 16)]
            av = iar[pl.ds(src, 16)]
            for c in range(cpg):
                ob[pl.ds(c * L + s, 16)] = (
                    plsc.load_gather(tP_ref, [ipv + c * V]) * scale)
                ob[pl.ds((cpg + c) * L + s, 16)] = (
                    plsc.load_gather(tF_ref, [ifv + c * V]) * scale)
                ob[pl.ds((2 * cpg + c) * L + s, 16)] = av
            return kc

        lax.fori_loop(0, nk, kbody, 0)

    def do_group(g, in_par, h, jj_guarded):
        gb0 = b0 + g * _G

        def jjbody(jj, jc):
            if jj_guarded:
                guard = jnp.logical_or(h > 0, jj > 0)
            j0 = 2 * jj

            def one(j, p):
                if jj_guarded:
                    @pl.when(guard)
                    def _():
                        drain_out(p)
                else:
                    drain_out(p)
                compute(j * L, in_par, p)
                fire_out(gb0 + j, p)

            one(j0, 0)
            one(j0 + 1, 1)
            return jc

        lax.fori_loop(0, _G // 2, jjbody, 0)

    fire_in(0, 0)

    def hbody(h, hc):
        g0 = 2 * h
        drain_in(0)
        fire_in(g0 + 1, 1)
        do_group(g0, 0, h, True)
        drain_in(1)

        @pl.when(h + 1 < ng // 2)
        def _():
            fire_in(g0 + 2, 0)

        do_group(g0 + 1, 1, h, False)
        return hc

    lax.fori_loop(0, ng // 2, hbody, 0)
    drain_out(0)
    drain_out(1)


def kernel(phoneme, a1, f2, phoneme_table, f2_table):
    B, L = phoneme.shape
    V, C = phoneme_table.shape
    scale = math.sqrt(C)
    info = plsc.get_sparse_core_info()
    nw = info.num_cores * info.num_subcores
    nbg = nw // _NCG
    cpg = C // _NCG
    nb = B // nbg
    assert C % _NCG == 0 and B % nbg == 0 and L % 8 == 0 and L >= 16
    assert nb % (2 * _G) == 0

    tPT = jnp.transpose(phoneme_table).reshape(-1)
    tFT = jnp.transpose(f2_table).reshape(-1)
    phoneme = phoneme.astype(jnp.int32).reshape(-1)
    f2 = f2.astype(jnp.int32).reshape(-1)
    a1 = a1.astype(jnp.float32).reshape(-1)

    mesh = plsc.VectorSubcoreMesh(core_axis_name="c", subcore_axis_name="s")
    run = pl.kernel(
        functools.partial(_emb_body, (B, L, V, C, nbg, cpg), scale),
        out_type=jax.ShapeDtypeStruct((B * 3 * C * L,), jnp.float32),
        mesh=mesh,
        compiler_params=pltpu.CompilerParams(needs_layout_passes=False),
        scratch_types=[
            pltpu.VMEM((cpg * V,), jnp.float32),
            pltpu.VMEM((cpg * V,), jnp.float32),
            pltpu.VMEM((_G * L,), jnp.int32),
            pltpu.VMEM((_G * L,), jnp.int32),
            pltpu.VMEM((_G * L,), jnp.int32),
            pltpu.VMEM((_G * L,), jnp.int32),
            pltpu.VMEM((_G * L,), jnp.float32),
            pltpu.VMEM((_G * L,), jnp.float32),
            pltpu.VMEM((3 * C // _NCG * L,), jnp.float32),
            pltpu.VMEM((3 * C // _NCG * L,), jnp.float32),
            pltpu.SemaphoreType.DMA,
            pltpu.SemaphoreType.DMA,
            pltpu.SemaphoreType.DMA,
            pltpu.SemaphoreType.DMA,
        ],
    )
    out = run(tPT, tFT, phoneme, a1, f2)
    return out.reshape(B, 3 * C, L)
